# Initial kernel scaffold; baseline (speedup 1.0000x reference)
import jax
import jax.numpy as jnp
from jax.experimental import pallas as pl


def kernel(x, fmx, fmy, fmz):
    raise NotImplementedError("write your pallas kernel here")


# speedup vs baseline: 1.0072x; 1.0072x over previous
import functools

import jax
import jax.numpy as jnp
from jax import lax
from jax.experimental import pallas as pl
from jax.experimental.pallas import tpu as pltpu
from jax.experimental.pallas import tpu_sc as plsc

NC = 2
NS = 16
NW = NC * NS
L = 16

B = 128


def _build(P, H, W, C):
    HW = H * W
    PPW = P // NW
    NCHUNK = PPW // B
    UV = ((1, 2), (0, 2), (0, 1))

    mesh = plsc.VectorSubcoreMesh(
        core_axis_name="c", subcore_axis_name="s",
        num_cores=NC, num_subcores=NS)

    @functools.partial(
        pl.kernel,
        out_type=jax.ShapeDtypeStruct((P, 3, C), jnp.float32),
        mesh=mesh,
        compiler_params=pltpu.CompilerParams(
            needs_layout_passes=False, use_tc_tiling_on_sc=False),
        scratch_types=[
            pltpu.VMEM((B,), jnp.float32),
            pltpu.VMEM((B,), jnp.float32),
            pltpu.VMEM((B,), jnp.float32),
            pltpu.VMEM((B,), jnp.int32),
            pltpu.VMEM((B,), jnp.int32),
            pltpu.VMEM((B,), jnp.int32),
            pltpu.VMEM((B,), jnp.int32),
            pltpu.VMEM((4 * B,), jnp.float32),
            pltpu.VMEM((B, C), jnp.float32),
            pltpu.VMEM((B, C), jnp.float32),
            pltpu.VMEM((B, C), jnp.float32),
            pltpu.VMEM((B, C), jnp.float32),
            pltpu.VMEM((B, 3, C), jnp.float32),
            pltpu.SemaphoreType.DMA,
        ],
    )
    def tri(xt_hbm, table_hbm, out_hbm,
            c0_v, c1_v, c2_v, i0_v, i1_v, i2_v, i3_v, w_v,
            r0_v, r1_v, r2_v, r3_v, o_v, sem):
        wid = lax.axis_index("c") * NS + lax.axis_index("s")
        wbase = wid * PPW
        c_refs = (c0_v, c1_v, c2_v)

        def chunk_body(t, carry):
            base = wbase + t * B
            for j in range(3):
                pltpu.sync_copy(xt_hbm.at[pl.ds(j * P + base, B)], c_refs[j])
            for q in range(3):
                uj, vj = UV[q]

                def iw_body(i, carry2):
                    s = pl.ds(i * L, L)
                    u = c_refs[uj][s]
                    v = c_refs[vj][s]
                    gu = (u + 1.0) * (0.5 * (W - 1))
                    gv = (v + 1.0) * (0.5 * (H - 1))
                    u0 = gu.astype(jnp.int32)
                    v0 = gv.astype(jnp.int32)
                    wx = gu - u0.astype(jnp.float32)
                    wy = gv - v0.astype(jnp.float32)
                    du = jnp.minimum(u0 + 1, W - 1) - u0
                    dv = (jnp.minimum(v0 + 1, H - 1) - v0) * W
                    base00 = v0 * W + u0 + q * HW
                    i0_v[s] = base00
                    i1_v[s] = base00 + du
                    i2_v[s] = base00 + dv
                    i3_v[s] = base00 + dv + du
                    w_v[pl.ds(0 * B + i * L, L)] = (1.0 - wx) * (1.0 - wy)
                    w_v[pl.ds(1 * B + i * L, L)] = wx * (1.0 - wy)
                    w_v[pl.ds(2 * B + i * L, L)] = (1.0 - wx) * wy
                    w_v[pl.ds(3 * B + i * L, L)] = wx * wy
                    return carry2

                lax.fori_loop(0, B // L, iw_body, 0)

                cps = [
                    pltpu.async_copy(table_hbm.at[i0_v], r0_v, sem),
                    pltpu.async_copy(table_hbm.at[i1_v], r1_v, sem),
                    pltpu.async_copy(table_hbm.at[i2_v], r2_v, sem),
                    pltpu.async_copy(table_hbm.at[i3_v], r3_v, sem),
                ]
                for cp in cps:
                    cp.wait()

                def fma_body(p, carry2):
                    pi = jnp.full((L,), p, dtype=jnp.int32)
                    w0 = plsc.load_gather(w_v, [pi])
                    w1 = plsc.load_gather(w_v, [pi + B])
                    w2 = plsc.load_gather(w_v, [pi + 2 * B])
                    w3 = plsc.load_gather(w_v, [pi + 3 * B])
                    for h in range(C // L):
                        sc = pl.ds(h * L, L)
                        o_v[p, q, sc] = (w0 * r0_v[p, sc] + w1 * r1_v[p, sc]
                                         + w2 * r2_v[p, sc] + w3 * r3_v[p, sc])
                    return carry2

                lax.fori_loop(0, B, fma_body, 0)

            pltpu.sync_copy(o_v, out_hbm.at[pl.ds(base, B)])
            return carry

        lax.fori_loop(0, NCHUNK, chunk_body, 0)

    return tri


def kernel(x, fmx, fmy, fmz):
    N, S, _ = x.shape
    C = fmx.shape[1]
    H, W = fmx.shape[2], fmx.shape[3]
    P = N * S

    planes = jnp.stack([fmx[0], fmy[0], fmz[0]], axis=0)
    table = planes.transpose(0, 2, 3, 1).reshape(3 * H * W, C)
    xt = x.reshape(P, 3).T.reshape(3 * P)

    out = _build(P, H, W, C)(xt, table)
    return out.reshape(N, S, 3, C)
